# baseline (speedup 1.0000x reference)
import functools

import jax
import jax.numpy as jnp
from jax import lax
from jax.experimental import pallas as pl
from jax.experimental.pallas import tpu as pltpu
from jax.experimental.pallas import tpu_sc as plsc

_D = 32
_S = 50
_BT = 128
_DT = _D // 8
_NP = _S // 2


@functools.cache
def _make_gather(n_seq: int):
    info = plsc.get_sparse_core_info()
    nw = info.num_cores * info.num_subcores
    nbt = n_seq // _BT
    bt_per_w = nbt // nw
    assert bt_per_w * nw == nbt
    spb = _S * _BT

    mesh = plsc.VectorSubcoreMesh(core_axis_name="c", subcore_axis_name="s")

    @functools.partial(
        pl.kernel,
        mesh=mesh,
        out_type=jax.ShapeDtypeStruct((_S, _DT, nbt, 8, _BT), jnp.float32),
        scratch_types=[
            pltpu.VMEM((spb,), jnp.int32),
            pltpu.VMEM((2 * _BT, _D), jnp.float32),
            pltpu.VMEM((2 * _BT, _D), jnp.float32),
            pltpu.VMEM((_DT, 8, _BT + 1), jnp.float32),
            pltpu.VMEM((_DT, 8, _BT + 1), jnp.float32),
            pltpu.VMEM((_DT, 8, _BT + 1), jnp.float32),
            pltpu.VMEM((_DT, 8, _BT + 1), jnp.float32),
            pltpu.SemaphoreType.DMA,
            pltpu.SemaphoreType.DMA,
            pltpu.SemaphoreType.DMA,
            pltpu.SemaphoreType.DMA,
        ],
        compiler_params=pltpu.CompilerParams(
            use_tc_tiling_on_sc=False, needs_layout_passes=False
        ),
    )
    def gather_kernel(idx_hbm, table_hbm, out_hbm, idx_v, rows_a, rows_b,
                      tile_a0, tile_a1, tile_b0, tile_b1, gsa, gsb, wsa, wsb):
        wid = lax.axis_index("s") * info.num_cores + lax.axis_index("c")
        iota = lax.iota(jnp.int32, 16)
        dt_inv = [iota >> 3, (iota + 16) >> 3]
        dl_inv = [iota & 7, (iota + 16) & 7]

        def start_gather(p, rows, sem):
            pltpu.async_copy(
                table_hbm.at[idx_v.at[pl.ds(p * 2 * _BT, 2 * _BT)]], rows, sem
            )

        def wait_gather(rows, sem):
            pltpu.make_async_copy(
                table_hbm.at[pl.ds(0, 2 * _BT)], rows, sem
            ).wait()

        def transpose(rows, q, tile):
            @pl.loop(0, _BT, unroll=32)
            def _(b):
                col = jnp.full((16,), b, jnp.int32)
                for h in range(2):
                    v = rows[q * _BT + b, pl.ds(h * 16, 16)]
                    plsc.store_scatter(tile, [dt_inv[h], dl_inv[h], col], v)

        def start_write(s, bt, tile, sem):
            pltpu.async_copy(
                tile.at[:, :, pl.ds(0, _BT)], out_hbm.at[s, :, bt], sem
            )

        def wait_write(tile, sem):
            pltpu.make_async_copy(
                out_hbm.at[0, :, 0], tile.at[:, :, pl.ds(0, _BT)], sem
            ).wait()

        def do_pair(p, bt, rows, t0, t1, wsem, first):
            @pl.when(jnp.logical_not(first))
            def _():
                wait_write(t0, wsem)
                wait_write(t1, wsem)

            transpose(rows, 0, t0)
            start_write(2 * p, bt, t0, wsem)
            transpose(rows, 1, t1)
            start_write(2 * p + 1, bt, t1, wsem)

        @pl.loop(0, bt_per_w)
        def _block(bb):
            bt = wid * bt_per_w + bb
            pltpu.sync_copy(idx_hbm.at[pl.ds(bt * spb, spb)], idx_v)
            start_gather(0, rows_a, gsa)

            @pl.loop(0, _NP // 2)
            def body(j):
                pa = 2 * j
                start_gather(pa + 1, rows_b, gsb)
                wait_gather(rows_a, gsa)
                do_pair(pa, bt, rows_a, tile_a0, tile_a1, wsa, j == 0)

                @pl.when(j < _NP // 2 - 1)
                def _():
                    start_gather(pa + 2, rows_a, gsa)

                wait_gather(rows_b, gsb)
                do_pair(pa + 1, bt, rows_b, tile_b0, tile_b1, wsb, j == 0)

            start_gather(_NP - 1, rows_a, gsa)
            wait_gather(rows_a, gsa)
            do_pair(_NP - 1, bt, rows_a, tile_a0, tile_a1, wsa, False)

            wait_write(tile_a0, wsa)
            wait_write(tile_a1, wsa)
            wait_write(tile_b0, wsb)
            wait_write(tile_b1, wsb)

    return gather_kernel


@jax.jit
def kernel(token_ids, weight):
    n_seq, s = token_ids.shape
    idx_r = (
        token_ids.astype(jnp.int32)
        .reshape(n_seq // _BT, _BT, s)
        .transpose(0, 2, 1)
        .reshape(-1)
    )
    out5 = _make_gather(n_seq)(idx_r, weight)
    return jnp.transpose(out5, (2, 4, 0, 1, 3)).reshape(n_seq, s, _D)


# speedup vs baseline: 1.1749x; 1.1749x over previous
import functools

import jax
import jax.numpy as jnp
from jax import lax
from jax.experimental import pallas as pl
from jax.experimental.pallas import tpu as pltpu
from jax.experimental.pallas import tpu_sc as plsc

_D = 32
_S = 50
_BT = 128
_DT = _D // 8
_NP = _S // 2


@functools.cache
def _make_gather(n_seq: int):
    info = plsc.get_sparse_core_info()
    nw = info.num_cores * info.num_subcores
    nbt = n_seq // _BT
    bt_per_w = nbt // nw
    assert bt_per_w * nw == nbt
    spb = _S * _BT

    mesh = plsc.VectorSubcoreMesh(core_axis_name="c", subcore_axis_name="s")

    @functools.partial(
        pl.kernel,
        mesh=mesh,
        out_type=jax.ShapeDtypeStruct((_S, _DT, nbt, 8, _BT), jnp.float32),
        scratch_types=[
            pltpu.VMEM((spb,), jnp.int32),
            pltpu.VMEM((2 * _BT, _D), jnp.float32),
            pltpu.VMEM((2 * _BT, _D), jnp.float32),
            pltpu.VMEM((_DT, 8, _BT + 1), jnp.float32),
            pltpu.VMEM((_DT, 8, _BT + 1), jnp.float32),
            pltpu.VMEM((_DT, 8, _BT + 1), jnp.float32),
            pltpu.VMEM((_DT, 8, _BT + 1), jnp.float32),
            pltpu.SemaphoreType.DMA,
            pltpu.SemaphoreType.DMA,
            pltpu.SemaphoreType.DMA,
            pltpu.SemaphoreType.DMA,
        ],
        compiler_params=pltpu.CompilerParams(
            use_tc_tiling_on_sc=False, needs_layout_passes=False
        ),
    )
    def gather_kernel(idx_hbm, table_hbm, out_hbm, idx_v, rows_a, rows_b,
                      tile_a0, tile_a1, tile_b0, tile_b1, gsa, gsb, wsa, wsb):
        wid = lax.axis_index("s") * info.num_cores + lax.axis_index("c")
        iota = lax.iota(jnp.int32, 16)
        dt_inv = [iota >> 3, (iota + 16) >> 3]
        dl_inv = [iota & 7, (iota + 16) & 7]

        def start_gather(p, rows, sem):
            pltpu.async_copy(
                table_hbm.at[idx_v.at[pl.ds(p * 2 * _BT, 2 * _BT)]], rows, sem
            )

        def wait_gather(rows, sem):
            pltpu.make_async_copy(
                table_hbm.at[pl.ds(0, 2 * _BT)], rows, sem
            ).wait()

        def transpose(rows, q, tile):
            @pl.loop(0, _BT, unroll=8)
            def _(b):
                col = jnp.full((16,), b, jnp.int32)
                for h in range(2):
                    v = rows[q * _BT + b, pl.ds(h * 16, 16)]
                    plsc.store_scatter(tile, [dt_inv[h], dl_inv[h], col], v)

        def start_write(s, bt, tile, sem):
            pltpu.async_copy(
                tile.at[:, :, pl.ds(0, _BT)], out_hbm.at[s, :, bt], sem
            )

        def wait_write(tile, sem):
            pltpu.make_async_copy(
                out_hbm.at[0, :, 0], tile.at[:, :, pl.ds(0, _BT)], sem
            ).wait()

        def do_pair(p, bt, rows, t0, t1, wsem, first):
            @pl.when(jnp.logical_not(first))
            def _():
                wait_write(t0, wsem)
                wait_write(t1, wsem)

            transpose(rows, 0, t0)
            start_write(2 * p, bt, t0, wsem)
            transpose(rows, 1, t1)
            start_write(2 * p + 1, bt, t1, wsem)

        @pl.loop(0, bt_per_w)
        def _block(bb):
            bt = wid * bt_per_w + bb
            pltpu.sync_copy(idx_hbm.at[pl.ds(bt * spb, spb)], idx_v)
            start_gather(0, rows_a, gsa)

            @pl.loop(0, _NP // 2)
            def body(j):
                pa = 2 * j
                start_gather(pa + 1, rows_b, gsb)
                wait_gather(rows_a, gsa)
                do_pair(pa, bt, rows_a, tile_a0, tile_a1, wsa, j == 0)

                @pl.when(j < _NP // 2 - 1)
                def _():
                    start_gather(pa + 2, rows_a, gsa)

                wait_gather(rows_b, gsb)
                do_pair(pa + 1, bt, rows_b, tile_b0, tile_b1, wsb, j == 0)

            start_gather(_NP - 1, rows_a, gsa)
            wait_gather(rows_a, gsa)
            do_pair(_NP - 1, bt, rows_a, tile_a0, tile_a1, wsa, False)

            wait_write(tile_a0, wsa)
            wait_write(tile_a1, wsa)
            wait_write(tile_b0, wsb)
            wait_write(tile_b1, wsb)

    return gather_kernel


@jax.jit
def kernel(token_ids, weight):
    n_seq, s = token_ids.shape
    idx_r = (
        token_ids.astype(jnp.int32)
        .reshape(n_seq // _BT, _BT, s)
        .transpose(0, 2, 1)
        .reshape(-1)
    )
    out5 = _make_gather(n_seq)(idx_r, weight)
    return jnp.transpose(out5, (2, 4, 0, 1, 3)).reshape(n_seq, s, _D)
